# baseline (speedup 1.0000x reference)
import functools
import math

import jax
import jax.numpy as jnp
from jax import lax
from jax.experimental import pallas as pl
from jax.experimental.pallas import tpu as pltpu
from jax.experimental.pallas import tpu_sc as plsc

_THRESHOLD = 0.8
_CAP_FACTOR = 1.25
_MIN_EXPERT_CAPACITY = 4
_LOSS_COEF = 0.01


def _routing_body(x_ref, wg_ref, coef_ref, pos_ref, xbuf_ref,
                  aux_ref, acc_ref, xsc_ref, *, T, TBLK, C, B, E):
    b = pl.program_id(0)
    i = pl.program_id(1)
    nblk = T // TBLK

    @pl.when(i == 0)
    def _init():
        acc_ref[0:3, :] = jnp.zeros((3, acc_ref.shape[1]), jnp.float32)
        xsc_ref[...] = jnp.zeros(xsc_ref.shape, xsc_ref.dtype)

        @pl.when(b == 0)
        def _init_aux():
            acc_ref[3:4, :] = jnp.zeros((1, acc_ref.shape[1]), jnp.float32)

    x_blk = x_ref[0]
    wg = wg_ref[...]
    logits = jnp.dot(x_blk, wg, preferred_element_type=jnp.float32)
    m = jnp.max(logits, axis=1, keepdims=True)
    ex = jnp.exp(logits - m)
    p = ex / jnp.sum(ex, axis=1, keepdims=True)

    EE = E * E
    rj = lax.broadcasted_iota(jnp.int32, (E, EE), 0)
    cj = lax.broadcasted_iota(jnp.int32, (E, EE), 1)
    repj = (cj // E == rj).astype(jnp.float32)
    repe = (cj % E == rj).astype(jnp.float32)
    hp = lax.Precision.HIGHEST
    p_j = lax.dot_general(p, repj, (((1,), (0,)), ((), ())),
                          preferred_element_type=jnp.float32, precision=hp)
    p_e = lax.dot_general(p, repe, (((1,), (0,)), ((), ())),
                          preferred_element_type=jnp.float32, precision=hp)
    cw = lax.broadcasted_iota(jnp.int32, (TBLK, EE), 1)
    jw = cw // E
    ew = cw % E
    gt = p_j > p_e
    eq = p_j == p_e
    rank_w = jnp.where(gt | (eq & (jw < ew)), 1.0, 0.0)
    csum_w = jnp.where(gt | (eq & (jw <= ew)), p_j, 0.0)
    r2 = lax.broadcasted_iota(jnp.int32, (EE, E), 0)
    c2 = lax.broadcasted_iota(jnp.int32, (EE, E), 1)
    red = (r2 % E == c2).astype(jnp.float32)
    rank = lax.dot_general(rank_w, red, (((1,), (0,)), ((), ())),
                           preferred_element_type=jnp.float32)
    csum = lax.dot_general(csum_w, red, (((1,), (0,)), ((), ())),
                           preferred_element_type=jnp.float32, precision=hp)
    keep = (csum < _THRESHOLD) | (rank == 0.0)
    maskf = keep.astype(jnp.float32)
    renorm = jnp.clip(jnp.sum(p * maskf, axis=1, keepdims=True), 1e-9, None)
    weight = p * maskf / renorm

    it0 = lax.broadcasted_iota(jnp.int32, (TBLK, TBLK), 0)
    it1 = lax.broadcasted_iota(jnp.int32, (TBLK, TBLK), 1)
    ltm = (it1 < it0).astype(jnp.float32)
    carry = acc_ref[0:1, 0:E]
    pos = lax.dot_general(
        ltm, maskf, (((1,), (0,)), ((), ())),
        preferred_element_type=jnp.float32,
    ) + carry
    acc_ref[0:1, 0:E] = carry + jnp.sum(maskf, axis=0, keepdims=True)
    keep_cap = (pos < float(C)) & keep
    kcf = keep_cap.astype(jnp.float32)
    coefs = weight * kcf
    iota_ef = lax.broadcasted_iota(jnp.int32, (TBLK, E), 1).astype(jnp.float32)
    coef_ref[0] = coefs
    pos_m = jnp.where(coefs > 0.0, jnp.minimum(pos, float(C - 1)), float(C))
    pos_ref[0] = pos_m.astype(jnp.int32)

    EC = E * C
    r3 = lax.broadcasted_iota(jnp.int32, (E, EC), 0)
    c3 = lax.broadcasted_iota(jnp.int32, (E, EC), 1)
    repc = (c3 // C == r3).astype(jnp.float32)
    pos_g = pos_m + float(C) * iota_ef
    rep_pos = lax.dot_general(pos_g, repc, (((1,), (0,)), ((), ())),
                              preferred_element_type=jnp.float32, precision=hp)
    rep_cf = lax.dot_general(coefs, repc, (((1,), (0,)), ((), ())),
                             preferred_element_type=jnp.float32, precision=hp)
    slotw = lax.broadcasted_iota(jnp.int32, (TBLK, EC), 1).astype(jnp.float32)
    oh_c = jnp.where(slotw == rep_pos, rep_cf, 0.0).astype(jnp.bfloat16)
    xb16 = x_blk.astype(jnp.bfloat16)
    xsc_ref[...] += lax.dot_general(
        oh_c, xb16, (((0,), (0,)), ((), ())),
        preferred_element_type=jnp.float32)

    @pl.when(i == nblk - 1)
    def _emit_xbuf():
        xbuf_ref[:, 0] = xsc_ref[...].reshape(E, C, xsc_ref.shape[-1]).astype(
            jnp.bfloat16)

    acc_ref[1:2, 0:E] = acc_ref[1:2, 0:E] + jnp.sum(maskf, axis=0, keepdims=True)
    acc_ref[2:3, 0:E] = acc_ref[2:3, 0:E] + jnp.sum(p, axis=0, keepdims=True)

    @pl.when(i == nblk - 1)
    def _finish_b():
        partial = jnp.sum(acc_ref[1:2, 0:E] * acc_ref[2:3, 0:E]).reshape(1, 1)
        acc_ref[3:4, 0:1] = acc_ref[3:4, 0:1] + partial

        @pl.when(b == B - 1)
        def _emit():
            scale = (E * E * _LOSS_COEF) / (float(T) * float(T) * B * E)
            aux_ref[0:1, 0:1] = acc_ref[3:4, 0:1] * scale


def _routing(x, w_gating, C):
    B, T, D = x.shape
    E = w_gating.shape[-1]
    TBLK = 512
    nblk = T // TBLK
    coef, pos, xbuf, aux = pl.pallas_call(
        functools.partial(_routing_body, T=T, TBLK=TBLK, C=C, B=B, E=E),
        grid=(B, nblk),
        in_specs=[
            pl.BlockSpec((1, TBLK, D), lambda b, i: (b, i, 0)),
            pl.BlockSpec((D, E), lambda b, i: (0, 0)),
        ],
        out_specs=[
            pl.BlockSpec((1, TBLK, E), lambda b, i: (b, i, 0)),
            pl.BlockSpec((1, TBLK, E), lambda b, i: (b, i, 0)),
            pl.BlockSpec((E, 1, C, D), lambda b, i: (0, b, 0, 0)),
            pl.BlockSpec((1, 1), lambda b, i: (0, 0)),
        ],
        out_shape=[
            jax.ShapeDtypeStruct((B, T, E), jnp.float32),
            jax.ShapeDtypeStruct((B, T, E), jnp.int32),
            jax.ShapeDtypeStruct((E, B, C, D), jnp.bfloat16),
            jax.ShapeDtypeStruct((1, 1), jnp.float32),
        ],
        scratch_shapes=[pltpu.VMEM((8, 128), jnp.float32),
                        pltpu.VMEM((E * C, D), jnp.float32)],
    )(x, w_gating)
    return coef, pos, xbuf, aux


def _make_dispatch(B, T, D, E, C):
    G = C // 2
    mesh = plsc.VectorSubcoreMesh(core_axis_name="c", subcore_axis_name="s")

    @functools.partial(
        pl.kernel,
        out_type=jax.ShapeDtypeStruct((E, B, C, D), jnp.float32),
        mesh=mesh,
        scratch_types=[
            pltpu.VMEM((2, G), jnp.int32),
            pltpu.VMEM((G, D), jnp.float32),
            pltpu.VMEM((G, D), jnp.float32),
            pltpu.SemaphoreType.DMA,
            pltpu.SemaphoreType.DMA,
            pltpu.SemaphoreType.DMA,
            pltpu.SemaphoreType.DMA,
        ],
    )
    def dispatch(tok_hbm, xflat_hbm, xbuf_hbm, tok_v, rows0_v, rows1_v,
                 sem0, sem1, sem2, sem3):
        b = lax.axis_index("c")
        e = lax.axis_index("s")
        pltpu.sync_copy(tok_hbm.at[b, e], tok_v)
        cp0 = pltpu.async_copy(xflat_hbm.at[tok_v.at[0]], rows0_v, sem0)
        cp1 = pltpu.async_copy(xflat_hbm.at[tok_v.at[1]], rows1_v, sem1)
        cp0.wait()
        wr0 = pltpu.async_copy(rows0_v, xbuf_hbm.at[e, b, pl.ds(0, G)], sem2)
        cp1.wait()
        wr1 = pltpu.async_copy(rows1_v, xbuf_hbm.at[e, b, pl.ds(G, G)], sem3)
        wr0.wait()
        wr1.wait()

    return dispatch


def _ffn_body(x_ref, w1_ref, w2_ref, out_ref):
    _, B, C, D = x_ref.shape
    xb = x_ref[0].reshape(B * C, D)
    w1b = w1_ref[0].astype(jnp.bfloat16)
    w2b = w2_ref[0].astype(jnp.bfloat16)
    h = jnp.dot(xb, w1b, preferred_element_type=jnp.float32)
    h = jnp.maximum(h, 0.0).astype(jnp.bfloat16)
    y = jnp.dot(h, w2b, preferred_element_type=jnp.float32)
    out_ref[0] = y.reshape(B, C, D).astype(jnp.bfloat16)


def _expert_ffn(xbuf, w1, w2):
    E, B, C, D = xbuf.shape
    H = w1.shape[-1]
    return pl.pallas_call(
        _ffn_body,
        grid=(E,),
        in_specs=[
            pl.BlockSpec((1, B, C, D), lambda e: (e, 0, 0, 0)),
            pl.BlockSpec((1, D, H), lambda e: (e, 0, 0)),
            pl.BlockSpec((1, H, D), lambda e: (e, 0, 0)),
        ],
        out_specs=pl.BlockSpec((1, B, C, D), lambda e: (e, 0, 0, 0)),
        out_shape=jax.ShapeDtypeStruct((E, B, C, D), jnp.bfloat16),
    )(xbuf, w1, w2)


def _combine_body(pos_ref, ebuf_ref, out_ref, *, C, E):
    TBLK2 = out_ref.shape[1]
    D = out_ref.shape[2]
    EC = E * C
    hp = lax.Precision.HIGHEST
    iota_ef = lax.broadcasted_iota(jnp.int32, (TBLK2, E), 1).astype(jnp.float32)
    pos_g = pos_ref[0].astype(jnp.float32) + float(C) * iota_ef
    r3 = lax.broadcasted_iota(jnp.int32, (E, EC), 0)
    c3 = lax.broadcasted_iota(jnp.int32, (E, EC), 1)
    repc = (c3 // C == r3).astype(jnp.float32)
    rep_pos = lax.dot_general(pos_g, repc, (((1,), (0,)), ((), ())),
                              preferred_element_type=jnp.float32, precision=hp)
    slotw = lax.broadcasted_iota(jnp.int32, (TBLK2, EC), 1).astype(jnp.float32)
    onehot = (slotw == rep_pos).astype(jnp.bfloat16)
    eflat = ebuf_ref[:, 0].reshape(EC, D)
    out_ref[0] = jnp.dot(onehot, eflat, preferred_element_type=jnp.float32)


def _combine(pos, ebuf):
    B, T, E = pos.shape
    _, _, C, D = ebuf.shape
    TBLK2 = 256
    nblk = T // TBLK2
    return pl.pallas_call(
        functools.partial(_combine_body, C=C, E=E),
        grid=(B, nblk),
        in_specs=[
            pl.BlockSpec((1, TBLK2, E), lambda b, i: (b, i, 0)),
            pl.BlockSpec((E, 1, C, D), lambda b, i: (0, b, 0, 0)),
        ],
        out_specs=pl.BlockSpec((1, TBLK2, D), lambda b, i: (b, i, 0)),
        out_shape=jax.ShapeDtypeStruct((B, T, D), jnp.float32),
    )(pos, ebuf)


def kernel(inputs, w_gating, w1, w2):
    B, T, D = inputs.shape
    E = w_gating.shape[-1]
    C = max(min(T, math.ceil(T * _CAP_FACTOR / E)), _MIN_EXPERT_CAPACITY)
    coef, pos, xbuf, aux = _routing(inputs, w_gating, C)
    ebuf = _expert_ffn(xbuf, w1, w2)
    out = _combine(pos, ebuf)
    return out, aux.reshape(())


# speedup vs baseline: 1.5451x; 1.5451x over previous
import functools
import math

import jax
import jax.numpy as jnp
from jax import lax
from jax.experimental import pallas as pl
from jax.experimental.pallas import tpu as pltpu
from jax.experimental.pallas import tpu_sc as plsc

_THRESHOLD = 0.8
_CAP_FACTOR = 1.25
_MIN_EXPERT_CAPACITY = 4
_LOSS_COEF = 0.01


def _routing_body(x_ref, wg_ref, coef_ref, pos_ref, xbuf_ref,
                  aux_ref, acc_ref, xsc_ref, *, T, TBLK, C, B, E):
    b = pl.program_id(0)
    i = pl.program_id(1)
    nblk = T // TBLK

    @pl.when(i == 0)
    def _init():
        acc_ref[0:3, :] = jnp.zeros((3, acc_ref.shape[1]), jnp.float32)
        xsc_ref[...] = jnp.zeros(xsc_ref.shape, xsc_ref.dtype)

        @pl.when(b == 0)
        def _init_aux():
            acc_ref[3:4, :] = jnp.zeros((1, acc_ref.shape[1]), jnp.float32)

    x_blk = x_ref[0]
    wg = wg_ref[...]
    logits = jnp.dot(x_blk, wg, preferred_element_type=jnp.float32)
    m = jnp.max(logits, axis=1, keepdims=True)
    ex = jnp.exp(logits - m)
    p = ex / jnp.sum(ex, axis=1, keepdims=True)

    EE = E * E
    rj = lax.broadcasted_iota(jnp.int32, (E, EE), 0)
    cj = lax.broadcasted_iota(jnp.int32, (E, EE), 1)
    repj = (cj // E == rj).astype(jnp.float32)
    repe = (cj % E == rj).astype(jnp.float32)
    hp = lax.Precision.HIGHEST
    p_j = lax.dot_general(p, repj, (((1,), (0,)), ((), ())),
                          preferred_element_type=jnp.float32, precision=hp)
    p_e = lax.dot_general(p, repe, (((1,), (0,)), ((), ())),
                          preferred_element_type=jnp.float32, precision=hp)
    cw = lax.broadcasted_iota(jnp.int32, (TBLK, EE), 1)
    jw = cw // E
    ew = cw % E
    gt = p_j > p_e
    eq = p_j == p_e
    rank_w = jnp.where(gt | (eq & (jw < ew)), 1.0, 0.0)
    csum_w = jnp.where(gt | (eq & (jw <= ew)), p_j, 0.0)
    r2 = lax.broadcasted_iota(jnp.int32, (EE, E), 0)
    c2 = lax.broadcasted_iota(jnp.int32, (EE, E), 1)
    red = (r2 % E == c2).astype(jnp.float32)
    rank = lax.dot_general(rank_w, red, (((1,), (0,)), ((), ())),
                           preferred_element_type=jnp.float32)
    csum = lax.dot_general(csum_w, red, (((1,), (0,)), ((), ())),
                           preferred_element_type=jnp.float32, precision=hp)
    keep = (csum < _THRESHOLD) | (rank == 0.0)
    maskf = keep.astype(jnp.float32)
    renorm = jnp.clip(jnp.sum(p * maskf, axis=1, keepdims=True), 1e-9, None)
    weight = p * maskf / renorm

    it0 = lax.broadcasted_iota(jnp.int32, (TBLK, TBLK), 0)
    it1 = lax.broadcasted_iota(jnp.int32, (TBLK, TBLK), 1)
    ltm = (it1 < it0).astype(jnp.float32)
    carry = acc_ref[0:1, 0:E]
    pos = lax.dot_general(
        ltm, maskf, (((1,), (0,)), ((), ())),
        preferred_element_type=jnp.float32,
    ) + carry
    acc_ref[0:1, 0:E] = carry + jnp.sum(maskf, axis=0, keepdims=True)
    keep_cap = (pos < float(C)) & keep
    kcf = keep_cap.astype(jnp.float32)
    coefs = weight * kcf
    coef_ref[0] = coefs
    pos_m = jnp.where(coefs > 0.0, jnp.minimum(pos, float(C - 1)), float(C))
    pos_ref[0] = pos_m.astype(jnp.int32)

    slot_i = lax.broadcasted_iota(jnp.int32, (TBLK, C), 1).astype(jnp.float32)
    xb16 = x_blk.astype(jnp.bfloat16)
    for e in range(E):
        oh_c = jnp.where(slot_i == pos[:, e:e + 1], coefs[:, e:e + 1], 0.0)
        xsc_ref[e] += lax.dot_general(
            oh_c.astype(jnp.bfloat16), xb16, (((0,), (0,)), ((), ())),
            preferred_element_type=jnp.float32)

    @pl.when(i == nblk - 1)
    def _emit_xbuf():
        xbuf_ref[:, 0] = xsc_ref[...].astype(jnp.bfloat16)

    acc_ref[1:2, 0:E] = acc_ref[1:2, 0:E] + jnp.sum(maskf, axis=0, keepdims=True)
    acc_ref[2:3, 0:E] = acc_ref[2:3, 0:E] + jnp.sum(p, axis=0, keepdims=True)

    @pl.when(i == nblk - 1)
    def _finish_b():
        partial = jnp.sum(acc_ref[1:2, 0:E] * acc_ref[2:3, 0:E]).reshape(1, 1)
        acc_ref[3:4, 0:1] = acc_ref[3:4, 0:1] + partial

        @pl.when(b == B - 1)
        def _emit():
            scale = (E * E * _LOSS_COEF) / (float(T) * float(T) * B * E)
            aux_ref[0:1, 0:1] = acc_ref[3:4, 0:1] * scale


def _routing(x, w_gating, C):
    B, T, D = x.shape
    E = w_gating.shape[-1]
    TBLK = 512
    nblk = T // TBLK
    coef, pos, xbuf, aux = pl.pallas_call(
        functools.partial(_routing_body, T=T, TBLK=TBLK, C=C, B=B, E=E),
        grid=(B, nblk),
        in_specs=[
            pl.BlockSpec((1, TBLK, D), lambda b, i: (b, i, 0)),
            pl.BlockSpec((D, E), lambda b, i: (0, 0)),
        ],
        out_specs=[
            pl.BlockSpec((1, TBLK, E), lambda b, i: (b, i, 0)),
            pl.BlockSpec((1, TBLK, E), lambda b, i: (b, i, 0)),
            pl.BlockSpec((E, 1, C, D), lambda b, i: (0, b, 0, 0)),
            pl.BlockSpec((1, 1), lambda b, i: (0, 0)),
        ],
        out_shape=[
            jax.ShapeDtypeStruct((B, T, E), jnp.float32),
            jax.ShapeDtypeStruct((B, T, E), jnp.int32),
            jax.ShapeDtypeStruct((E, B, C, D), jnp.bfloat16),
            jax.ShapeDtypeStruct((1, 1), jnp.float32),
        ],
        scratch_shapes=[pltpu.VMEM((8, 128), jnp.float32),
                        pltpu.VMEM((E, C, D), jnp.float32)],
    )(x, w_gating)
    return coef, pos, xbuf, aux


def _make_dispatch(B, T, D, E, C):
    G = C // 2
    mesh = plsc.VectorSubcoreMesh(core_axis_name="c", subcore_axis_name="s")

    @functools.partial(
        pl.kernel,
        out_type=jax.ShapeDtypeStruct((E, B, C, D), jnp.float32),
        mesh=mesh,
        scratch_types=[
            pltpu.VMEM((2, G), jnp.int32),
            pltpu.VMEM((G, D), jnp.float32),
            pltpu.VMEM((G, D), jnp.float32),
            pltpu.SemaphoreType.DMA,
            pltpu.SemaphoreType.DMA,
            pltpu.SemaphoreType.DMA,
            pltpu.SemaphoreType.DMA,
        ],
    )
    def dispatch(tok_hbm, xflat_hbm, xbuf_hbm, tok_v, rows0_v, rows1_v,
                 sem0, sem1, sem2, sem3):
        b = lax.axis_index("c")
        e = lax.axis_index("s")
        pltpu.sync_copy(tok_hbm.at[b, e], tok_v)
        cp0 = pltpu.async_copy(xflat_hbm.at[tok_v.at[0]], rows0_v, sem0)
        cp1 = pltpu.async_copy(xflat_hbm.at[tok_v.at[1]], rows1_v, sem1)
        cp0.wait()
        wr0 = pltpu.async_copy(rows0_v, xbuf_hbm.at[e, b, pl.ds(0, G)], sem2)
        cp1.wait()
        wr1 = pltpu.async_copy(rows1_v, xbuf_hbm.at[e, b, pl.ds(G, G)], sem3)
        wr0.wait()
        wr1.wait()

    return dispatch


def _ffn_body(x_ref, w1_ref, w2_ref, out_ref):
    _, B, C, D = x_ref.shape
    xb = x_ref[0].reshape(B * C, D)
    w1b = w1_ref[0].astype(jnp.bfloat16)
    w2b = w2_ref[0].astype(jnp.bfloat16)
    h = jnp.dot(xb, w1b, preferred_element_type=jnp.float32)
    h = jnp.maximum(h, 0.0).astype(jnp.bfloat16)
    y = jnp.dot(h, w2b, preferred_element_type=jnp.float32)
    out_ref[0] = y.reshape(B, C, D).astype(jnp.bfloat16)


def _expert_ffn(xbuf, w1, w2):
    E, B, C, D = xbuf.shape
    H = w1.shape[-1]
    return pl.pallas_call(
        _ffn_body,
        grid=(E,),
        in_specs=[
            pl.BlockSpec((1, B, C, D), lambda e: (e, 0, 0, 0)),
            pl.BlockSpec((1, D, H), lambda e: (e, 0, 0)),
            pl.BlockSpec((1, H, D), lambda e: (e, 0, 0)),
        ],
        out_specs=pl.BlockSpec((1, B, C, D), lambda e: (e, 0, 0, 0)),
        out_shape=jax.ShapeDtypeStruct((E, B, C, D), jnp.bfloat16),
    )(xbuf, w1, w2)


def _combine_body(pos_ref, ebuf_ref, out_ref, *, C, E):
    TBLK2 = out_ref.shape[1]
    D = out_ref.shape[2]
    slot_i = lax.broadcasted_iota(jnp.int32, (TBLK2, C), 1)
    acc = jnp.zeros((TBLK2, D), jnp.float32)
    for e in range(E):
        pcol = pos_ref[0, :, e:e + 1]
        onehot = (slot_i == pcol).astype(jnp.bfloat16)
        acc = acc + jnp.dot(onehot, ebuf_ref[e, 0],
                            preferred_element_type=jnp.float32)
    out_ref[0] = acc


def _combine(pos, ebuf):
    B, T, E = pos.shape
    _, _, C, D = ebuf.shape
    TBLK2 = 512
    nblk = T // TBLK2
    return pl.pallas_call(
        functools.partial(_combine_body, C=C, E=E),
        grid=(B, nblk),
        in_specs=[
            pl.BlockSpec((1, TBLK2, E), lambda b, i: (b, i, 0)),
            pl.BlockSpec((E, 1, C, D), lambda b, i: (0, b, 0, 0)),
        ],
        out_specs=pl.BlockSpec((1, TBLK2, D), lambda b, i: (b, i, 0)),
        out_shape=jax.ShapeDtypeStruct((B, T, D), jnp.float32),
    )(pos, ebuf)


def kernel(inputs, w_gating, w1, w2):
    B, T, D = inputs.shape
    E = w_gating.shape[-1]
    C = max(min(T, math.ceil(T * _CAP_FACTOR / E)), _MIN_EXPERT_CAPACITY)
    coef, pos, xbuf, aux = _routing(inputs, w_gating, C)
    ebuf = _expert_ffn(xbuf, w1, w2)
    out = _combine(pos, ebuf)
    return out, aux.reshape(())


# speedup vs baseline: 1.5472x; 1.0014x over previous
import functools
import math

import jax
import jax.numpy as jnp
from jax import lax
from jax.experimental import pallas as pl
from jax.experimental.pallas import tpu as pltpu

_THRESHOLD = 0.8
_CAP_FACTOR = 1.25
_MIN_EXPERT_CAPACITY = 4
_LOSS_COEF = 0.01


def _routing_body(x_ref, wg_ref, coef_ref, pos_ref, xbuf_ref,
                  aux_ref, acc_ref, xsc_ref, *, T, TBLK, C, B, E):
    b = pl.program_id(0)
    i = pl.program_id(1)
    nblk = T // TBLK

    @pl.when(i == 0)
    def _init():
        acc_ref[0:3, :] = jnp.zeros((3, acc_ref.shape[1]), jnp.float32)
        xsc_ref[...] = jnp.zeros(xsc_ref.shape, xsc_ref.dtype)

        @pl.when(b == 0)
        def _init_aux():
            acc_ref[3:4, :] = jnp.zeros((1, acc_ref.shape[1]), jnp.float32)

    x_blk = x_ref[0]
    wg = wg_ref[...]
    logits = jnp.dot(x_blk, wg, preferred_element_type=jnp.float32)
    m = jnp.max(logits, axis=1, keepdims=True)
    ex = jnp.exp(logits - m)
    p = ex / jnp.sum(ex, axis=1, keepdims=True)

    EE = E * E
    rj = lax.broadcasted_iota(jnp.int32, (E, EE), 0)
    cj = lax.broadcasted_iota(jnp.int32, (E, EE), 1)
    repj = (cj // E == rj).astype(jnp.float32)
    repe = (cj % E == rj).astype(jnp.float32)
    hp = lax.Precision.HIGHEST
    p_j = lax.dot_general(p, repj, (((1,), (0,)), ((), ())),
                          preferred_element_type=jnp.float32, precision=hp)
    p_e = lax.dot_general(p, repe, (((1,), (0,)), ((), ())),
                          preferred_element_type=jnp.float32, precision=hp)
    cw = lax.broadcasted_iota(jnp.int32, (TBLK, EE), 1)
    jw = cw // E
    ew = cw % E
    gt = p_j > p_e
    eq = p_j == p_e
    rank_w = jnp.where(gt | (eq & (jw < ew)), 1.0, 0.0)
    csum_w = jnp.where(gt | (eq & (jw <= ew)), p_j, 0.0)
    r2 = lax.broadcasted_iota(jnp.int32, (EE, E), 0)
    c2 = lax.broadcasted_iota(jnp.int32, (EE, E), 1)
    red = (r2 % E == c2).astype(jnp.float32)
    rank = lax.dot_general(rank_w, red, (((1,), (0,)), ((), ())),
                           preferred_element_type=jnp.float32)
    csum = lax.dot_general(csum_w, red, (((1,), (0,)), ((), ())),
                           preferred_element_type=jnp.float32, precision=hp)
    keep = (csum < _THRESHOLD) | (rank == 0.0)
    maskf = keep.astype(jnp.float32)
    renorm = jnp.clip(jnp.sum(p * maskf, axis=1, keepdims=True), 1e-9, None)
    weight = p * maskf / renorm

    it0 = lax.broadcasted_iota(jnp.int32, (TBLK, TBLK), 0)
    it1 = lax.broadcasted_iota(jnp.int32, (TBLK, TBLK), 1)
    ltm = (it1 < it0).astype(jnp.float32)
    carry = acc_ref[0:1, 0:E]
    pos = lax.dot_general(
        ltm, maskf, (((1,), (0,)), ((), ())),
        preferred_element_type=jnp.float32,
    ) + carry
    acc_ref[0:1, 0:E] = carry + jnp.sum(maskf, axis=0, keepdims=True)
    keep_cap = (pos < float(C)) & keep
    kcf = keep_cap.astype(jnp.float32)
    coefs = weight * kcf
    coef_ref[0] = coefs
    pos_m = jnp.where(coefs > 0.0, jnp.minimum(pos, float(C - 1)), float(C))
    pos_ref[0] = pos_m.astype(jnp.int32)

    slot_i = lax.broadcasted_iota(jnp.int32, (TBLK, C), 1).astype(jnp.float32)
    xb16 = x_blk.astype(jnp.bfloat16)
    for e in range(E):
        oh_c = jnp.where(slot_i == pos[:, e:e + 1], coefs[:, e:e + 1], 0.0)
        xsc_ref[e] += lax.dot_general(
            oh_c.astype(jnp.bfloat16), xb16, (((0,), (0,)), ((), ())),
            preferred_element_type=jnp.float32)

    @pl.when(i == nblk - 1)
    def _emit_xbuf():
        xbuf_ref[:, 0] = xsc_ref[...].astype(jnp.bfloat16)

    acc_ref[1:2, 0:E] = acc_ref[1:2, 0:E] + jnp.sum(maskf, axis=0, keepdims=True)
    acc_ref[2:3, 0:E] = acc_ref[2:3, 0:E] + jnp.sum(p, axis=0, keepdims=True)

    @pl.when(i == nblk - 1)
    def _finish_b():
        partial = jnp.sum(acc_ref[1:2, 0:E] * acc_ref[2:3, 0:E]).reshape(1, 1)
        acc_ref[3:4, 0:1] = acc_ref[3:4, 0:1] + partial

        @pl.when(b == B - 1)
        def _emit():
            scale = (E * E * _LOSS_COEF) / (float(T) * float(T) * B * E)
            aux_ref[0:1, 0:1] = acc_ref[3:4, 0:1] * scale


def _routing(x, w_gating, C):
    B, T, D = x.shape
    E = w_gating.shape[-1]
    TBLK = 512
    nblk = T // TBLK
    coef, pos, xbuf, aux = pl.pallas_call(
        functools.partial(_routing_body, T=T, TBLK=TBLK, C=C, B=B, E=E),
        grid=(B, nblk),
        in_specs=[
            pl.BlockSpec((1, TBLK, D), lambda b, i: (b, i, 0)),
            pl.BlockSpec((D, E), lambda b, i: (0, 0)),
        ],
        out_specs=[
            pl.BlockSpec((1, TBLK, E), lambda b, i: (b, i, 0)),
            pl.BlockSpec((1, TBLK, E), lambda b, i: (b, i, 0)),
            pl.BlockSpec((E, 1, C, D), lambda b, i: (0, b, 0, 0)),
            pl.BlockSpec((1, 1), lambda b, i: (0, 0)),
        ],
        out_shape=[
            jax.ShapeDtypeStruct((B, T, E), jnp.float32),
            jax.ShapeDtypeStruct((B, T, E), jnp.int32),
            jax.ShapeDtypeStruct((E, B, C, D), jnp.bfloat16),
            jax.ShapeDtypeStruct((1, 1), jnp.float32),
        ],
        scratch_shapes=[pltpu.VMEM((8, 128), jnp.float32),
                        pltpu.VMEM((E, C, D), jnp.float32)],
    )(x, w_gating)
    return coef, pos, xbuf, aux


def _ffn_body(x_ref, w1_ref, w2_ref, out_ref):
    _, B, C, D = x_ref.shape
    xb = x_ref[0].reshape(B * C, D)
    w1b = w1_ref[0].astype(jnp.bfloat16)
    w2b = w2_ref[0].astype(jnp.bfloat16)
    h = jnp.dot(xb, w1b, preferred_element_type=jnp.float32)
    h = jnp.maximum(h, 0.0).astype(jnp.bfloat16)
    y = jnp.dot(h, w2b, preferred_element_type=jnp.float32)
    out_ref[0] = y.reshape(B, C, D).astype(jnp.bfloat16)


def _expert_ffn(xbuf, w1, w2):
    E, B, C, D = xbuf.shape
    H = w1.shape[-1]
    return pl.pallas_call(
        _ffn_body,
        grid=(E,),
        in_specs=[
            pl.BlockSpec((1, B, C, D), lambda e: (e, 0, 0, 0)),
            pl.BlockSpec((1, D, H), lambda e: (e, 0, 0)),
            pl.BlockSpec((1, H, D), lambda e: (e, 0, 0)),
        ],
        out_specs=pl.BlockSpec((1, B, C, D), lambda e: (e, 0, 0, 0)),
        out_shape=jax.ShapeDtypeStruct((E, B, C, D), jnp.bfloat16),
    )(xbuf, w1, w2)


def _combine_body(pos_ref, ebuf_ref, out_ref, *, C, E):
    TBLK2 = out_ref.shape[1]
    D = out_ref.shape[2]
    slot_i = lax.broadcasted_iota(jnp.int32, (TBLK2, C), 1)
    acc = jnp.zeros((TBLK2, D), jnp.float32)
    for e in range(E):
        pcol = pos_ref[0, :, e:e + 1]
        onehot = (slot_i == pcol).astype(jnp.bfloat16)
        acc = acc + jnp.dot(onehot, ebuf_ref[e, 0],
                            preferred_element_type=jnp.float32)
    out_ref[0] = acc


def _combine(pos, ebuf):
    B, T, E = pos.shape
    _, _, C, D = ebuf.shape
    TBLK2 = 512
    nblk = T // TBLK2
    return pl.pallas_call(
        functools.partial(_combine_body, C=C, E=E),
        grid=(B, nblk),
        in_specs=[
            pl.BlockSpec((1, TBLK2, E), lambda b, i: (b, i, 0)),
            pl.BlockSpec((E, 1, C, D), lambda b, i: (0, b, 0, 0)),
        ],
        out_specs=pl.BlockSpec((1, TBLK2, D), lambda b, i: (b, i, 0)),
        out_shape=jax.ShapeDtypeStruct((B, T, D), jnp.float32),
    )(pos, ebuf)


def kernel(inputs, w_gating, w1, w2):
    B, T, D = inputs.shape
    E = w_gating.shape[-1]
    C = max(min(T, math.ceil(T * _CAP_FACTOR / E)), _MIN_EXPERT_CAPACITY)
    coef, pos, xbuf, aux = _routing(inputs, w_gating, C)
    ebuf = _expert_ffn(xbuf, w1, w2)
    out = _combine(pos, ebuf)
    return out, aux.reshape(())
